# baseline (speedup 1.0000x reference)
import jax
import jax.numpy as jnp
from jax.experimental import pallas as pl
from jax.experimental.pallas import tpu as pltpu


def kernel(x, row_emb, col_emb):
    b = x.shape[0]
    h, w = x.shape[-2], x.shape[-1]
    d = row_emb.shape[1]

    def body(col_ref, row_ref, out_ref, s4, sem):
        colT = col_ref[:w, :].T
        rowT = row_ref[:h, :].T
        for j in range(h):
            s4[0:d, j, :] = colT
            s4[d:2 * d, j, :] = jnp.broadcast_to(rowT[:, j][:, None], (d, w))
        copies = [
            pltpu.make_async_copy(s4, out_ref.at[i], sem) for i in range(b)
        ]
        for c in copies:
            c.start()
        for c in copies:
            c.wait()

    out = pl.pallas_call(
        body,
        out_specs=pl.BlockSpec(memory_space=pl.ANY),
        out_shape=jax.ShapeDtypeStruct((b, 2 * d, h, w), jnp.float32),
        scratch_shapes=[
            pltpu.VMEM((2 * d, h, w), jnp.float32),
            pltpu.SemaphoreType.DMA,
        ],
    )(col_emb, row_emb)
    return out


# speedup vs baseline: 11.4122x; 11.4122x over previous
import jax
import jax.numpy as jnp
from jax.experimental import pallas as pl
from jax.experimental.pallas import tpu as pltpu

_BPG = 2


def kernel(x, row_emb, col_emb):
    b = x.shape[0]
    h, w = x.shape[-2], x.shape[-1]
    d = row_emb.shape[1]

    def body(col_ref, row_ref, out_ref):
        col = col_ref[:w, :]
        row = row_ref[:h, :]
        out_ref[:, :, :, 0:d] = jnp.broadcast_to(
            col[None, None, :, :], (_BPG, h, w, d))
        out_ref[:, :, :, d:2 * d] = jnp.broadcast_to(
            row[None, :, None, :], (_BPG, h, w, d))

    out = pl.pallas_call(
        body,
        grid=(b // _BPG,),
        in_specs=[
            pl.BlockSpec(col_emb.shape, lambda i: (0, 0)),
            pl.BlockSpec(row_emb.shape, lambda i: (0, 0)),
        ],
        out_specs=pl.BlockSpec((_BPG, h, w, 2 * d), lambda i: (i, 0, 0, 0)),
        out_shape=jax.ShapeDtypeStruct((b, h, w, 2 * d), jnp.float32),
    )(col_emb, row_emb)
    return jnp.transpose(out, (0, 3, 1, 2))
